# baseline (speedup 1.0000x reference)
import jax
import jax.numpy as jnp
from jax import lax
from jax.experimental import pallas as pl
from jax.experimental.pallas import tpu as pltpu

_NUM_UNITS = 500
_SPEECH_DIM = 256
_HIDDEN = 1024
_LLM_DIM = 2048
_VPAD = 512
_BB = 8

_INV_SQRT2 = 0.7071067811865476


def _body(ids_ref, e_ref, w1_ref, b1_ref, w2_ref, b2_ref, g_ref, bt_ref,
          o_ref):
    rows = o_ref.shape[0] * o_ref.shape[1]
    ids = ids_ref[0]
    cols = lax.broadcasted_iota(jnp.int32, (rows, _VPAD), 1)
    onehot = (cols == ids).astype(jnp.bfloat16)
    x = jnp.dot(onehot, e_ref[...], preferred_element_type=jnp.float32)
    h = jnp.dot(x.astype(jnp.bfloat16), w1_ref[...],
                preferred_element_type=jnp.float32)
    h = h + b1_ref[...]
    h = 0.5 * h * (1.0 + lax.erf(h * _INV_SQRT2))
    y = jnp.dot(h.astype(jnp.bfloat16), w2_ref[...],
                preferred_element_type=jnp.float32)
    y = y + b2_ref[...]
    mu = jnp.mean(y, axis=-1, keepdims=True)
    yc = y - mu
    var = jnp.mean(yc * yc, axis=-1, keepdims=True)
    out = yc * lax.rsqrt(var + 1e-5) * g_ref[...] + bt_ref[...]
    o_ref[...] = out.reshape(o_ref.shape)


@jax.jit
def kernel(local_ids, embed_W, W1, b1, W2, b2, ln_gamma, ln_beta):
    B, T = local_ids.shape
    grid = B // _BB
    rows = T * _BB
    ids = (local_ids.astype(jnp.int32).T
           .reshape(T, grid, _BB).transpose(1, 0, 2).reshape(grid, rows, 1))
    e_pad = jnp.zeros((_VPAD, _SPEECH_DIM), jnp.bfloat16).at[:_NUM_UNITS].set(
        embed_W.astype(jnp.bfloat16))
    w1 = W1.astype(jnp.bfloat16)
    w2 = W2.astype(jnp.bfloat16)

    full = lambda *shape: pl.BlockSpec(shape, lambda i: (0,) * len(shape))
    out = pl.pallas_call(
        _body,
        grid=(grid,),
        in_specs=[
            pl.BlockSpec((1, rows, 1), lambda i: (i, 0, 0)),
            full(_VPAD, _SPEECH_DIM),
            full(_SPEECH_DIM, _HIDDEN),
            full(1, _HIDDEN),
            full(_HIDDEN, _LLM_DIM),
            full(1, _LLM_DIM),
            full(1, _LLM_DIM),
            full(1, _LLM_DIM),
        ],
        out_specs=pl.BlockSpec((T, _BB, _LLM_DIM), lambda i: (0, i, 0)),
        out_shape=jax.ShapeDtypeStruct((T, B, _LLM_DIM), jnp.float32),
        compiler_params=pltpu.CompilerParams(
            dimension_semantics=("arbitrary",)),
    )(ids, e_pad, w1, b1.reshape(1, _HIDDEN), w2, b2.reshape(1, _LLM_DIM),
      ln_gamma.reshape(1, _LLM_DIM), ln_beta.reshape(1, _LLM_DIM))
    return out.transpose(1, 0, 2)


# speedup vs baseline: 1.0711x; 1.0711x over previous
import jax
import jax.numpy as jnp
from jax import lax
from jax.experimental import pallas as pl
from jax.experimental.pallas import tpu as pltpu

_NUM_UNITS = 500
_SPEECH_DIM = 256
_HIDDEN = 1024
_LLM_DIM = 2048
_VPAD = 512
_BB = 16

_INV_SQRT2 = 0.7071067811865476


def _body(ids_ref, e_ref, w1_ref, b1_ref, w2_ref, b2_ref, g_ref, bt_ref,
          o_ref):
    rows = o_ref.shape[0] * o_ref.shape[1]
    ids = ids_ref[0]
    cols = lax.broadcasted_iota(jnp.int32, (rows, _VPAD), 1)
    onehot = (cols == ids).astype(jnp.float32)
    x = jnp.dot(onehot, e_ref[...], preferred_element_type=jnp.float32)
    h = jnp.dot(x, w1_ref[...], preferred_element_type=jnp.float32)
    h = h + b1_ref[...]
    h = 0.5 * h * (1.0 + lax.erf(h * _INV_SQRT2))
    y = jnp.dot(h, w2_ref[...], preferred_element_type=jnp.float32)
    y = y + b2_ref[...]
    mu = jnp.mean(y, axis=-1, keepdims=True)
    yc = y - mu
    var = jnp.mean(yc * yc, axis=-1, keepdims=True)
    out = yc * lax.rsqrt(var + 1e-5) * g_ref[...] + bt_ref[...]
    o_ref[...] = out.reshape(o_ref.shape)


@jax.jit
def kernel(local_ids, embed_W, W1, b1, W2, b2, ln_gamma, ln_beta):
    B, T = local_ids.shape
    grid = B // _BB
    rows = T * _BB
    ids = (local_ids.astype(jnp.int32).T
           .reshape(T, grid, _BB).transpose(1, 0, 2).reshape(grid, rows, 1))
    e_pad = jnp.zeros((_VPAD, _SPEECH_DIM), jnp.float32).at[:_NUM_UNITS].set(
        embed_W)
    w1 = W1
    w2 = W2

    full = lambda *shape: pl.BlockSpec(shape, lambda i: (0,) * len(shape))
    out = pl.pallas_call(
        _body,
        grid=(grid,),
        in_specs=[
            pl.BlockSpec((1, rows, 1), lambda i: (i, 0, 0)),
            full(_VPAD, _SPEECH_DIM),
            full(_SPEECH_DIM, _HIDDEN),
            full(1, _HIDDEN),
            full(_HIDDEN, _LLM_DIM),
            full(1, _LLM_DIM),
            full(1, _LLM_DIM),
            full(1, _LLM_DIM),
        ],
        out_specs=pl.BlockSpec((T, _BB, _LLM_DIM), lambda i: (0, i, 0)),
        out_shape=jax.ShapeDtypeStruct((T, B, _LLM_DIM), jnp.float32),
        compiler_params=pltpu.CompilerParams(
            dimension_semantics=("arbitrary",)),
    )(ids, e_pad, w1, b1.reshape(1, _HIDDEN), w2, b2.reshape(1, _LLM_DIM),
      ln_gamma.reshape(1, _LLM_DIM), ln_beta.reshape(1, _LLM_DIM))
    return out.transpose(1, 0, 2)
